# baseline (speedup 1.0000x reference)
import jax
import jax.numpy as jnp
from jax.experimental import pallas as pl

N = 10000
R = 4
B = 512


def _silu(v):
    return v * jax.nn.sigmoid(v)


def _bn(v, p):
    m = jnp.mean(v, axis=0)
    var = jnp.var(v, axis=0)
    return (v - m) / jnp.sqrt(var + 1e-5) * p["g"] + p["b"]


def _rgcn_conv(x, src, dst, edge_type, p):
    out = x @ p["w_root"] + p["b"]
    xs = jnp.take(x, src, axis=0)
    for r in range(R):
        mask = (edge_type == r).astype(x.dtype)
        msg = xs * mask[:, None]
        agg = jnp.zeros((x.shape[0], x.shape[1]), x.dtype).at[dst].add(msg)
        cnt = jnp.zeros((x.shape[0],), x.dtype).at[dst].add(mask)
        agg = agg / jnp.maximum(cnt, 1.0)[:, None]
        out = out + agg @ p["w_rel"][r]
    return out


def _head_kernel(hg_ref, w0_ref, b0_ref, w1_ref, b1_ref, w2_ref, b2_ref, o_ref):
    h = hg_ref[...]
    h = h @ w0_ref[...] + b0_ref[...]
    h = h * jax.nn.sigmoid(h)
    h = h @ w1_ref[...] + b1_ref[...]
    h = h * jax.nn.sigmoid(h)
    o_ref[...] = h @ w2_ref[...] + b2_ref[...]


def kernel(x, edge_index, edge_type, batch, mol_feats, params):
    src = edge_index[0]
    dst = edge_index[1]
    h = _silu(_bn(_rgcn_conv(x, src, dst, edge_type, params["gc"][0]), params["bn_gc"]))
    h = _silu(_rgcn_conv(h, src, dst, edge_type, params["gc"][1]))
    h = _silu(_rgcn_conv(h, src, dst, edge_type, params["gc"][2]))
    hg = jax.ops.segment_sum(h, batch, num_segments=B)
    h_m = _silu(_bn(mol_feats @ params["fc_m"][0]["w"] + params["fc_m"][0]["b"], params["bn_m"]))
    h_m = _silu(h_m @ params["fc_m"][1]["w"] + params["fc_m"][1]["b"])
    hg = jnp.concatenate([hg, h_m], axis=1)
    fc = params["fc"]
    out = pl.pallas_call(
        _head_kernel,
        out_shape=jax.ShapeDtypeStruct((B, 1), jnp.float32),
    )(hg, fc[0]["w"], fc[0]["b"], fc[1]["w"], fc[1]["b"], fc[2]["w"], fc[2]["b"])
    return out
